# baseline (speedup 1.0000x reference)
import functools

import jax
import jax.numpy as jnp
from jax import lax
from jax.experimental import pallas as pl
from jax.experimental.pallas import tpu as pltpu
from jax.experimental.pallas import tpu_sc as plsc

B, S, F, V, D = 4096, 20, 26, 100000, 32
N = B * S * F
L = 16
NC, NS = 2, 16
NW = NC * NS
ROWS_PER_W = N // NW
C = 1664
CHUNKS = ROWS_PER_W // C
NBUF = 2

VC = 768
VFULL = V // VC
VTAIL = V - VFULL * VC
T_TASKS = F * VFULL + F
T_ROUNDS = -(-T_TASKS // NW)


@functools.cache
def _build_transpose():
    mesh = plsc.VectorSubcoreMesh(core_axis_name="c", subcore_axis_name="s")

    @functools.partial(
        pl.kernel,
        mesh=mesh,
        out_type=jax.ShapeDtypeStruct((F * V * D,), jnp.float32),
        scratch_types=[
            pltpu.VMEM((D, VC), jnp.float32),
            pltpu.VMEM((VC * D,), jnp.float32),
            pltpu.VMEM((VTAIL, D), jnp.float32),
        ],
        compiler_params=pltpu.CompilerParams(use_tc_tiling_on_sc=True,
                                             needs_layout_passes=False),
    )
    def _transpose_kernel(tabt_hbm, tail_hbm, flat_hbm, stage_v, tr_v, stg2_v):
        wid = lax.axis_index("s") * NC + lax.axis_index("c")
        iota = lax.iota(jnp.int32, L)
        col = iota * D

        def round_body(r, _):
            cid = r * NW + wid

            @pl.when(cid < F * VFULL)
            def _full():
                f = cid // VFULL
                v0 = (cid % VFULL) * VC
                pltpu.sync_copy(tabt_hbm.at[f, :, pl.ds(v0, VC)], stage_v)

                def g_body(g, _):
                    base = col + g * (L * D)
                    for d in range(D):
                        x = stage_v[d, pl.ds(g * L, L)]
                        plsc.store_scatter(tr_v, [base + d], x)
                    return 0

                lax.fori_loop(0, VC // L, g_body, 0, unroll=False)
                pltpu.sync_copy(tr_v,
                                flat_hbm.at[pl.ds((f * V + v0) * D, VC * D)])

            @pl.when(jnp.logical_and(cid >= F * VFULL, cid < T_TASKS))
            def _tail():
                f = cid - F * VFULL
                pltpu.sync_copy(tail_hbm.at[f], stg2_v)

                def t_body(g, _):
                    row, c = g // 2, g % 2
                    tr_v[pl.ds(g * L, L)] = stg2_v[row, pl.ds(c * L, L)]
                    return 0

                lax.fori_loop(0, VTAIL * D // L, t_body, 0, unroll=False)
                pltpu.sync_copy(
                    tr_v.at[pl.ds(0, VTAIL * D)],
                    flat_hbm.at[pl.ds((f * V + VFULL * VC) * D, VTAIL * D)])

            return 0

        lax.fori_loop(0, T_ROUNDS, round_body, 0, unroll=False)

    return _transpose_kernel


@functools.cache
def _build():
    mesh = plsc.VectorSubcoreMesh(core_axis_name="c", subcore_axis_name="s")

    @functools.partial(
        pl.kernel,
        mesh=mesh,
        out_type=jax.ShapeDtypeStruct((N, D), jnp.float32),
        scratch_types=[
            pltpu.VMEM((C,), jnp.int32),
            pltpu.VMEM((C,), jnp.int32),
            pltpu.VMEM((C, D), jnp.float32),
            pltpu.VMEM((C, D), jnp.float32),
            pltpu.VMEM((C,), jnp.int32),
            pltpu.SemaphoreType.DMA,
            pltpu.SemaphoreType.DMA,
            pltpu.SemaphoreType.DMA,
            pltpu.SemaphoreType.DMA,
        ],
        compiler_params=pltpu.CompilerParams(use_tc_tiling_on_sc=False),
    )
    def _gather_kernel(idx_hbm, tab_hbm, out_hbm, i0, i1, r0, r1, pat,
                       gs0, gs1, os0, os1):
        idxv, rowsv = [i0, i1], [r0, r1]
        gsem, osem = [gs0, gs1], [os0, os1]
        wid = lax.axis_index("s") * NC + lax.axis_index("c")
        wbase = wid * ROWS_PER_W

        def pat_body(j, _):
            pos = j * L + lax.iota(jnp.int32, L)
            pat[pl.ds(j * L, L)] = lax.rem(pos, F) * V
            return 0

        lax.fori_loop(0, C // L, pat_body, 0, unroll=False)

        def prep_idx(g, b):
            base = wbase + g * C
            pltpu.sync_copy(idx_hbm.at[pl.ds(base, C)], idxv[b])

            def add_body(j, _):
                sl = pl.ds(j * L, L)
                idxv[b][sl] = idxv[b][sl] + pat[sl]
                return 0

            lax.fori_loop(0, C // L, add_body, 0, unroll=False)

        def issue_gather(b):
            pltpu.async_copy(tab_hbm.at[idxv[b]], rowsv[b], gsem[b])

        def finish(g, b):
            pltpu.make_async_copy(tab_hbm.at[idxv[b]], rowsv[b], gsem[b]).wait()
            base = wbase + g * C
            pltpu.async_copy(rowsv[b], out_hbm.at[pl.ds(base, C)], osem[b])

        def wait_out(g, b):
            base = wbase + g * C
            pltpu.make_async_copy(
                rowsv[b], out_hbm.at[pl.ds(base, C)], osem[b]).wait()

        for b in range(NBUF):
            prep_idx(b, b)
            issue_gather(b)

        def loop_body(g0, _):
            for b in range(NBUF):
                g = g0 * NBUF + b
                finish(g, b)
                prep_idx(g + NBUF, b)
                wait_out(g, b)
                issue_gather(b)
            return 0

        lax.fori_loop(0, CHUNKS // NBUF - 1, loop_body, 0, unroll=False)

        for b in range(NBUF):
            g = CHUNKS - NBUF + b
            finish(g, b)
            wait_out(g, b)

    return _gather_kernel


def kernel(inputs, tables):
    idx_flat = inputs.reshape(N)
    tabt = jnp.transpose(tables, (0, 2, 1))
    tail = tables[:, VFULL * VC:, :]
    flat1d = _build_transpose()(tabt, tail)
    tab_flat = flat1d.reshape(F * V, D)
    out = _build()(idx_flat, tab_flat)
    return out.reshape(B, S, F, D)


# speedup vs baseline: 1.0976x; 1.0976x over previous
import functools

import jax
import jax.numpy as jnp
from jax import lax
from jax.experimental import pallas as pl
from jax.experimental.pallas import tpu as pltpu
from jax.experimental.pallas import tpu_sc as plsc

B, S, F, V, D = 4096, 20, 26, 100000, 32
N = B * S * F
L = 16
NC, NS = 2, 16
NW = NC * NS
ROWS_PER_W = N // NW
C = 1664
CHUNKS = ROWS_PER_W // C
NBUF = 2

VC = 768
VFULL = V // VC
VTAIL = V - VFULL * VC
T_TASKS = F * VFULL + F
T_ROUNDS = -(-T_TASKS // NW)


NFULL = F * VFULL
FR = -(-NFULL // NW)


@functools.cache
def _build_transpose():
    mesh = plsc.VectorSubcoreMesh(core_axis_name="c", subcore_axis_name="s")

    @functools.partial(
        pl.kernel,
        mesh=mesh,
        out_type=jax.ShapeDtypeStruct((F * V * D,), jnp.float32),
        scratch_types=[
            pltpu.VMEM((D, VC), jnp.float32),
            pltpu.VMEM((D, VC), jnp.float32),
            pltpu.VMEM((VC * D,), jnp.float32),
            pltpu.VMEM((VC * D,), jnp.float32),
            pltpu.VMEM((VTAIL, D), jnp.float32),
            pltpu.SemaphoreType.DMA,
            pltpu.SemaphoreType.DMA,
            pltpu.SemaphoreType.DMA,
            pltpu.SemaphoreType.DMA,
        ],
        compiler_params=pltpu.CompilerParams(use_tc_tiling_on_sc=True,
                                             needs_layout_passes=False),
    )
    def _transpose_kernel(tabt_hbm, tail_hbm, flat_hbm, s0, s1, t0, t1,
                          stg2_v, i0, i1, o0, o1):
        stage, tr = [s0, s1], [t0, t1]
        isem, osem = [i0, i1], [o0, o1]
        wid = lax.axis_index("s") * NC + lax.axis_index("c")
        col = lax.iota(jnp.int32, L) * D
        cols = [col + k for k in range(8)]

        def in_copy(cid, b):
            f = cid // VFULL
            v0 = (cid % VFULL) * VC
            return pltpu.make_async_copy(
                tabt_hbm.at[f, :, pl.ds(v0, VC)], stage[b], isem[b])

        def out_copy(cid, b):
            f = cid // VFULL
            v0 = (cid % VFULL) * VC
            return pltpu.make_async_copy(
                tr[b], flat_hbm.at[pl.ds((f * V + v0) * D, VC * D)], osem[b])

        def compute(b):
            def g_body(g, _):
                base = g * (L * D)
                for d in range(D):
                    x = stage[b][d, pl.ds(g * L, L)]
                    plsc.store_scatter(
                        tr[b].at[pl.ds(base + (d & ~7), (L - 1) * D + 8)],
                        [cols[d % 8]], x)
                return 0

            lax.fori_loop(0, VC // L, g_body, 0, unroll=False)

        @pl.when(wid < NFULL)
        def _prime():
            in_copy(wid, 0).start()

        def dbl_body(r2, _):
            for b in range(2):
                r = r2 * 2 + b
                cid = r * NW + wid
                ncid = cid + NW

                @pl.when(ncid < NFULL)
                def _start_next():
                    in_copy(ncid, 1 - b).start()

                @pl.when(cid < NFULL)
                def _work():
                    in_copy(cid, b).wait()
                    pcid = cid - 2 * NW

                    @pl.when(pcid >= 0)
                    def _drain_prev():
                        out_copy(pcid, b).wait()

                    compute(b)
                    out_copy(cid, b).start()

            return 0

        lax.fori_loop(0, FR // 2, dbl_body, 0, unroll=False)

        r_max = (NFULL - 1 - wid) // NW
        for b in range(2):
            x_b = r_max - ((r_max - b) % 2)
            out_copy(x_b * NW + wid, b).wait()

        @pl.when(wid < F)
        def _tail():
            pltpu.sync_copy(tail_hbm.at[wid], stg2_v)

            def t_body(g, _):
                row, c = g // 2, g % 2
                t0[pl.ds(g * L, L)] = stg2_v[row, pl.ds(c * L, L)]
                return 0

            lax.fori_loop(0, VTAIL * D // L, t_body, 0, unroll=False)
            pltpu.sync_copy(
                t0.at[pl.ds(0, VTAIL * D)],
                flat_hbm.at[pl.ds((wid * V + VFULL * VC) * D, VTAIL * D)])

    return _transpose_kernel


@functools.cache
def _build():
    mesh = plsc.VectorSubcoreMesh(core_axis_name="c", subcore_axis_name="s")

    @functools.partial(
        pl.kernel,
        mesh=mesh,
        out_type=jax.ShapeDtypeStruct((N, D), jnp.float32),
        scratch_types=[
            pltpu.VMEM((C,), jnp.int32),
            pltpu.VMEM((C,), jnp.int32),
            pltpu.VMEM((C, D), jnp.float32),
            pltpu.VMEM((C, D), jnp.float32),
            pltpu.VMEM((C,), jnp.int32),
            pltpu.SemaphoreType.DMA,
            pltpu.SemaphoreType.DMA,
            pltpu.SemaphoreType.DMA,
            pltpu.SemaphoreType.DMA,
        ],
        compiler_params=pltpu.CompilerParams(use_tc_tiling_on_sc=False),
    )
    def _gather_kernel(idx_hbm, tab_hbm, out_hbm, i0, i1, r0, r1, pat,
                       gs0, gs1, os0, os1):
        idxv, rowsv = [i0, i1], [r0, r1]
        gsem, osem = [gs0, gs1], [os0, os1]
        wid = lax.axis_index("s") * NC + lax.axis_index("c")
        wbase = wid * ROWS_PER_W

        def pat_body(j, _):
            pos = j * L + lax.iota(jnp.int32, L)
            pat[pl.ds(j * L, L)] = lax.rem(pos, F) * V
            return 0

        lax.fori_loop(0, C // L, pat_body, 0, unroll=False)

        def prep_idx(g, b):
            base = wbase + g * C
            pltpu.sync_copy(idx_hbm.at[pl.ds(base, C)], idxv[b])

            def add_body(j, _):
                sl = pl.ds(j * L, L)
                idxv[b][sl] = idxv[b][sl] + pat[sl]
                return 0

            lax.fori_loop(0, C // L, add_body, 0, unroll=False)

        def issue_gather(b):
            pltpu.async_copy(tab_hbm.at[idxv[b]], rowsv[b], gsem[b])

        def finish(g, b):
            pltpu.make_async_copy(tab_hbm.at[idxv[b]], rowsv[b], gsem[b]).wait()
            base = wbase + g * C
            pltpu.async_copy(rowsv[b], out_hbm.at[pl.ds(base, C)], osem[b])

        def wait_out(g, b):
            base = wbase + g * C
            pltpu.make_async_copy(
                rowsv[b], out_hbm.at[pl.ds(base, C)], osem[b]).wait()

        for b in range(NBUF):
            prep_idx(b, b)
            issue_gather(b)

        def loop_body(g0, _):
            for b in range(NBUF):
                g = g0 * NBUF + b
                finish(g, b)
                prep_idx(g + NBUF, b)
                wait_out(g, b)
                issue_gather(b)
            return 0

        lax.fori_loop(0, CHUNKS // NBUF - 1, loop_body, 0, unroll=False)

        for b in range(NBUF):
            g = CHUNKS - NBUF + b
            finish(g, b)
            wait_out(g, b)

    return _gather_kernel


def kernel(inputs, tables):
    idx_flat = inputs.reshape(N)
    tabt = jnp.transpose(tables, (0, 2, 1))
    tail = tables[:, VFULL * VC:, :]
    flat1d = _build_transpose()(tabt, tail)
    tab_flat = flat1d.reshape(F * V, D)
    out = _build()(idx_flat, tab_flat)
    return out.reshape(B, S, F, D)


# speedup vs baseline: 1.2848x; 1.1705x over previous
import functools

import jax
import jax.numpy as jnp
from jax import lax
from jax.experimental import pallas as pl
from jax.experimental.pallas import tpu as pltpu
from jax.experimental.pallas import tpu_sc as plsc

B, S, F, V, D = 4096, 20, 26, 100000, 32
N = B * S * F
L = 16
NC, NS = 2, 16
NW = NC * NS
ROWS_PER_W = N // NW
C = 1664
CHUNKS = ROWS_PER_W // C
NBUF = 2

VC = 768
VFULL = V // VC
VTAIL = V - VFULL * VC
T_TASKS = F * VFULL + F
T_ROUNDS = -(-T_TASKS // NW)


NFULL = F * VFULL
FR = -(-NFULL // NW)


@functools.cache
def _build_transpose():
    mesh = plsc.VectorSubcoreMesh(core_axis_name="c", subcore_axis_name="s")

    @functools.partial(
        pl.kernel,
        mesh=mesh,
        out_type=jax.ShapeDtypeStruct((F * V * D,), jnp.float32),
        scratch_types=[
            pltpu.VMEM((D, VC), jnp.float32),
            pltpu.VMEM((D, VC), jnp.float32),
            pltpu.VMEM((VC * D,), jnp.float32),
            pltpu.VMEM((VC * D,), jnp.float32),
            pltpu.VMEM((VTAIL, D), jnp.float32),
            pltpu.SemaphoreType.DMA,
            pltpu.SemaphoreType.DMA,
            pltpu.SemaphoreType.DMA,
            pltpu.SemaphoreType.DMA,
        ],
        compiler_params=pltpu.CompilerParams(use_tc_tiling_on_sc=True,
                                             needs_layout_passes=False),
    )
    def _transpose_kernel(tabt_hbm, tail_hbm, flat_hbm, s0, s1, t0, t1,
                          stg2_v, i0, i1, o0, o1):
        stage, tr = [s0, s1], [t0, t1]
        isem, osem = [i0, i1], [o0, o1]
        wid = lax.axis_index("s") * NC + lax.axis_index("c")
        col = lax.iota(jnp.int32, L) * D
        cols = [col + k for k in range(8)]

        def in_copy(cid, b):
            f = cid // VFULL
            v0 = (cid % VFULL) * VC
            return pltpu.make_async_copy(
                tabt_hbm.at[f, :, pl.ds(v0, VC)], stage[b], isem[b])

        def out_copy(cid, b):
            f = cid // VFULL
            v0 = (cid % VFULL) * VC
            return pltpu.make_async_copy(
                tr[b], flat_hbm.at[pl.ds((f * V + v0) * D, VC * D)], osem[b])

        def compute(b):
            @plsc.parallel_loop(0, VC // L)
            def g_body(g):
                base = g * (L * D)
                for d in range(D):
                    x = stage[b][d, pl.ds(g * L, L)]
                    plsc.store_scatter(
                        tr[b].at[pl.ds(base + (d & ~7), (L - 1) * D + 8)],
                        [cols[d % 8]], x)

        @pl.when(wid < NFULL)
        def _prime():
            in_copy(wid, 0).start()

        def dbl_body(r2, _):
            for b in range(2):
                r = r2 * 2 + b
                cid = r * NW + wid
                ncid = cid + NW

                @pl.when(ncid < NFULL)
                def _start_next():
                    in_copy(ncid, 1 - b).start()

                @pl.when(cid < NFULL)
                def _work():
                    in_copy(cid, b).wait()
                    pcid = cid - 2 * NW

                    @pl.when(pcid >= 0)
                    def _drain_prev():
                        out_copy(pcid, b).wait()

                    compute(b)
                    out_copy(cid, b).start()

            return 0

        lax.fori_loop(0, FR // 2, dbl_body, 0, unroll=False)

        r_max = (NFULL - 1 - wid) // NW
        for b in range(2):
            x_b = r_max - ((r_max - b) % 2)
            out_copy(x_b * NW + wid, b).wait()

        @pl.when(wid < F)
        def _tail():
            pltpu.sync_copy(tail_hbm.at[wid], stg2_v)

            def t_body(g, _):
                row, c = g // 2, g % 2
                t0[pl.ds(g * L, L)] = stg2_v[row, pl.ds(c * L, L)]
                return 0

            lax.fori_loop(0, VTAIL * D // L, t_body, 0, unroll=False)
            pltpu.sync_copy(
                t0.at[pl.ds(0, VTAIL * D)],
                flat_hbm.at[pl.ds((wid * V + VFULL * VC) * D, VTAIL * D)])

    return _transpose_kernel


@functools.cache
def _build():
    mesh = plsc.VectorSubcoreMesh(core_axis_name="c", subcore_axis_name="s")

    @functools.partial(
        pl.kernel,
        mesh=mesh,
        out_type=jax.ShapeDtypeStruct((N, D), jnp.float32),
        scratch_types=[
            pltpu.VMEM((C,), jnp.int32),
            pltpu.VMEM((C,), jnp.int32),
            pltpu.VMEM((C, D), jnp.float32),
            pltpu.VMEM((C, D), jnp.float32),
            pltpu.VMEM((C,), jnp.int32),
            pltpu.SemaphoreType.DMA,
            pltpu.SemaphoreType.DMA,
            pltpu.SemaphoreType.DMA,
            pltpu.SemaphoreType.DMA,
        ],
        compiler_params=pltpu.CompilerParams(use_tc_tiling_on_sc=False),
    )
    def _gather_kernel(idx_hbm, tab_hbm, out_hbm, i0, i1, r0, r1, pat,
                       gs0, gs1, os0, os1):
        idxv, rowsv = [i0, i1], [r0, r1]
        gsem, osem = [gs0, gs1], [os0, os1]
        wid = lax.axis_index("s") * NC + lax.axis_index("c")
        wbase = wid * ROWS_PER_W

        def pat_body(j, _):
            pos = j * L + lax.iota(jnp.int32, L)
            pat[pl.ds(j * L, L)] = lax.rem(pos, F) * V
            return 0

        lax.fori_loop(0, C // L, pat_body, 0, unroll=False)

        def prep_idx(g, b):
            base = wbase + g * C
            pltpu.sync_copy(idx_hbm.at[pl.ds(base, C)], idxv[b])

            def add_body(j, _):
                sl = pl.ds(j * L, L)
                idxv[b][sl] = idxv[b][sl] + pat[sl]
                return 0

            lax.fori_loop(0, C // L, add_body, 0, unroll=False)

        def issue_gather(b):
            pltpu.async_copy(tab_hbm.at[idxv[b]], rowsv[b], gsem[b])

        def finish(g, b):
            pltpu.make_async_copy(tab_hbm.at[idxv[b]], rowsv[b], gsem[b]).wait()
            base = wbase + g * C
            pltpu.async_copy(rowsv[b], out_hbm.at[pl.ds(base, C)], osem[b])

        def wait_out(g, b):
            base = wbase + g * C
            pltpu.make_async_copy(
                rowsv[b], out_hbm.at[pl.ds(base, C)], osem[b]).wait()

        for b in range(NBUF):
            prep_idx(b, b)
            issue_gather(b)

        def loop_body(g0, _):
            for b in range(NBUF):
                g = g0 * NBUF + b
                finish(g, b)
                prep_idx(g + NBUF, b)
                wait_out(g, b)
                issue_gather(b)
            return 0

        lax.fori_loop(0, CHUNKS // NBUF - 1, loop_body, 0, unroll=False)

        for b in range(NBUF):
            g = CHUNKS - NBUF + b
            finish(g, b)
            wait_out(g, b)

    return _gather_kernel


def kernel(inputs, tables):
    idx_flat = inputs.reshape(N)
    tabt = jnp.transpose(tables, (0, 2, 1))
    tail = tables[:, VFULL * VC:, :]
    flat1d = _build_transpose()(tabt, tail)
    tab_flat = flat1d.reshape(F * V, D)
    out = _build()(idx_flat, tab_flat)
    return out.reshape(B, S, F, D)


# speedup vs baseline: 2.5735x; 2.0030x over previous
import functools

import jax
import jax.numpy as jnp
from jax import lax
from jax.experimental import pallas as pl
from jax.experimental.pallas import tpu as pltpu
from jax.experimental.pallas import tpu_sc as plsc

B, S, F, V, D = 4096, 20, 26, 100000, 32
L = 16
NC, NS = 2, 16
NW = NC * NS

AC = 4992
AFULL = 99840 // AC
ATAIL = V - AFULL * AC
NT = F * AFULL


@functools.cache
def _build_detile():
    mesh = plsc.VectorSubcoreMesh(core_axis_name="c", subcore_axis_name="s")

    @functools.partial(
        pl.kernel,
        mesh=mesh,
        out_type=jax.ShapeDtypeStruct((F * D * V,), jnp.float32),
        scratch_types=[
            pltpu.VMEM((AC,), jnp.float32),
            pltpu.VMEM((AC,), jnp.float32),
            pltpu.VMEM((ATAIL, D), jnp.float32),
            pltpu.VMEM((ATAIL * D,), jnp.float32),
            pltpu.SemaphoreType.DMA,
            pltpu.SemaphoreType.DMA,
            pltpu.SemaphoreType.DMA,
            pltpu.SemaphoreType.DMA,
        ],
        compiler_params=pltpu.CompilerParams(use_tc_tiling_on_sc=True,
                                             needs_layout_passes=False),
    )
    def _detile_kernel(tabt_hbm, tail_hbm, flat_hbm, c0, c1, stg2, tr2,
                       i0, i1, o0, o1):
        buf, isem, osem = [c0, c1], [i0, i1], [o0, o1]
        wid = lax.axis_index("s") * NC + lax.axis_index("c")
        d = wid

        def in_copy(t, b):
            f, c = t // AFULL, t % AFULL
            return pltpu.make_async_copy(
                tabt_hbm.at[f, d, pl.ds(c * AC, AC)], buf[b], isem[b])

        def out_copy(t, b):
            f, c = t // AFULL, t % AFULL
            base = (f * D + d) * V + c * AC
            return pltpu.make_async_copy(
                buf[b], flat_hbm.at[pl.ds(base, AC)], osem[b])

        in_copy(0, 0).start()

        def dbl_body(t2, _):
            for b in range(2):
                t = t2 * 2 + b

                @pl.when(t + 1 < NT)
                def _start_next():
                    in_copy(t + 1, 1 - b).start()

                in_copy(t, b).wait()

                @pl.when(t - 2 >= 0)
                def _drain_prev():
                    out_copy(t - 2, b).wait()

                out_copy(t, b).start()
            return 0

        lax.fori_loop(0, NT // 2, dbl_body, 0, unroll=False)
        for b in range(2):
            out_copy(NT - 2 + b, b).wait()

        @pl.when(wid < F)
        def _tail():
            f = wid
            pltpu.sync_copy(tail_hbm.at[f], stg2)
            colv = lax.iota(jnp.int32, L) * ATAIL

            def t_body(j, _):
                x0 = stg2[j, pl.ds(0, L)]
                x1 = stg2[j, pl.ds(L, L)]
                plsc.store_scatter(tr2, [colv + j], x0)
                plsc.store_scatter(
                    tr2.at[pl.ds(L * ATAIL, L * ATAIL)], [colv + j], x1)
                return 0

            lax.fori_loop(0, ATAIL, t_body, 0, unroll=False)

            def t_out(dd, _):
                base = (f * D + dd) * V + AFULL * AC
                pltpu.sync_copy(tr2.at[pl.ds(dd * ATAIL, ATAIL)],
                                flat_hbm.at[pl.ds(base, ATAIL)])
                return 0

            lax.fori_loop(0, D, t_out, 0, unroll=False)

    return _detile_kernel


@functools.cache
def _build_lookup():
    mesh = plsc.VectorSubcoreMesh(core_axis_name="c", subcore_axis_name="s")

    @functools.partial(
        pl.kernel,
        mesh=mesh,
        out_type=jax.ShapeDtypeStruct((S, F, 4, B // 128, 8, 128),
                                      jnp.float32),
        scratch_types=[
            pltpu.VMEM((V,), jnp.float32),
            pltpu.VMEM((B,), jnp.int32),
            pltpu.VMEM((B,), jnp.int32),
            pltpu.VMEM((B // 128, 128), jnp.float32),
            pltpu.VMEM((B // 128, 128), jnp.float32),
            pltpu.SemaphoreType.DMA,
            pltpu.SemaphoreType.DMA,
            pltpu.SemaphoreType.DMA,
            pltpu.SemaphoreType.DMA,
        ],
        compiler_params=pltpu.CompilerParams(needs_layout_passes=False),
    )
    def _lookup_kernel(idx_hbm, tab_hbm, out_hbm, rowbuf, ib0, ib1, ob0, ob1,
                       is0, is1, os0, os1):
        ibuf, obuf = [ib0, ib1], [ob0, ob1]
        isem, osem = [is0, is1], [os0, os1]
        wid = lax.axis_index("s") * NC + lax.axis_index("c")
        d = wid
        dt, di = d // 8, d % 8

        def idx_copy(f, s, b):
            return pltpu.make_async_copy(
                idx_hbm.at[pl.ds((f * S + s) * B, B)], ibuf[b], isem[b])

        def out_cp(f, s, b):
            return pltpu.make_async_copy(
                obuf[b], out_hbm.at[s, f, dt, :, di, :], osem[b])

        def f_body(f, _):
            pltpu.sync_copy(tab_hbm.at[pl.ds((f * D + d) * V, V)], rowbuf)
            idx_copy(f, 0, 0).start()

            def s_dbl(s2, _):
                for b in range(2):
                    s = s2 * 2 + b

                    @pl.when(s + 1 < S)
                    def _prefetch():
                        idx_copy(f, s + 1, 1 - b).start()

                    idx_copy(f, s, b).wait()

                    @pl.when(s - 2 >= 0)
                    def _drain():
                        out_cp(f, s - 2, b).wait()

                    @plsc.parallel_loop(0, B // L)
                    def g_body(j):
                        iv = ibuf[b][pl.ds(j * L, L)]
                        x = plsc.load_gather(rowbuf, [iv])
                        obuf[b][j // 8, pl.ds((j % 8) * L, L)] = x

                    out_cp(f, s, b).start()
                return 0

            lax.fori_loop(0, S // 2, s_dbl, 0, unroll=False)
            for b in range(2):
                out_cp(f, S - 2 + b, b).wait()
            return 0

        lax.fori_loop(0, F, f_body, 0, unroll=False)

    return _lookup_kernel


def kernel(inputs, tables):
    tabt = jnp.transpose(tables, (0, 2, 1))
    tail = tables[:, AFULL * AC:, :]
    flat = _build_detile()(tabt, tail)
    idx1 = jnp.transpose(inputs, (2, 1, 0)).reshape(F * S * B)
    out6 = _build_lookup()(idx1, flat)
    return jnp.transpose(out6, (3, 5, 0, 1, 2, 4)).reshape(B, S, F, D)


# speedup vs baseline: 3.8836x; 1.5091x over previous
import functools

import jax
import jax.numpy as jnp
from jax import lax
from jax.experimental import pallas as pl
from jax.experimental.pallas import tpu as pltpu
from jax.experimental.pallas import tpu_sc as plsc

B, S, F, V, D = 4096, 20, 26, 100000, 32
L = 16
NC, NS = 2, 16
NW = NC * NS

AC = 4992
AFULL = 99840 // AC
ATAIL = V - AFULL * AC
NT = F * AFULL


@functools.cache
def _build_detile():
    mesh = plsc.VectorSubcoreMesh(core_axis_name="c", subcore_axis_name="s")

    @functools.partial(
        pl.kernel,
        mesh=mesh,
        out_type=jax.ShapeDtypeStruct((F * D * V,), jnp.float32),
        scratch_types=[
            pltpu.VMEM((AC,), jnp.float32),
            pltpu.VMEM((AC,), jnp.float32),
            pltpu.VMEM((ATAIL, D), jnp.float32),
            pltpu.VMEM((ATAIL * D,), jnp.float32),
            pltpu.SemaphoreType.DMA,
            pltpu.SemaphoreType.DMA,
            pltpu.SemaphoreType.DMA,
            pltpu.SemaphoreType.DMA,
        ],
        compiler_params=pltpu.CompilerParams(use_tc_tiling_on_sc=True,
                                             needs_layout_passes=False),
    )
    def _detile_kernel(tabt_hbm, tail_hbm, flat_hbm, c0, c1, stg2, tr2,
                       i0, i1, o0, o1):
        buf, isem, osem = [c0, c1], [i0, i1], [o0, o1]
        wid = lax.axis_index("s") * NC + lax.axis_index("c")
        d = wid

        def in_copy(t, b):
            f, c = t // AFULL, t % AFULL
            return pltpu.make_async_copy(
                tabt_hbm.at[f, d, pl.ds(c * AC, AC)], buf[b], isem[b])

        def out_copy(t, b):
            f, c = t // AFULL, t % AFULL
            base = (f * D + d) * V + c * AC
            return pltpu.make_async_copy(
                buf[b], flat_hbm.at[pl.ds(base, AC)], osem[b])

        in_copy(0, 0).start()

        def dbl_body(t2, _):
            for b in range(2):
                t = t2 * 2 + b

                @pl.when(t + 1 < NT)
                def _start_next():
                    in_copy(t + 1, 1 - b).start()

                in_copy(t, b).wait()

                @pl.when(t - 2 >= 0)
                def _drain_prev():
                    out_copy(t - 2, b).wait()

                out_copy(t, b).start()
            return 0

        lax.fori_loop(0, NT // 2, dbl_body, 0, unroll=False)
        for b in range(2):
            out_copy(NT - 2 + b, b).wait()

        @pl.when(wid < F)
        def _tail():
            f = wid
            pltpu.sync_copy(tail_hbm.at[f], stg2)
            colv = lax.iota(jnp.int32, L) * ATAIL

            def t_body(j, _):
                x0 = stg2[j, pl.ds(0, L)]
                x1 = stg2[j, pl.ds(L, L)]
                plsc.store_scatter(tr2, [colv + j], x0)
                plsc.store_scatter(
                    tr2.at[pl.ds(L * ATAIL, L * ATAIL)], [colv + j], x1)
                return 0

            lax.fori_loop(0, ATAIL, t_body, 0, unroll=False)

            def t_out(dd, _):
                base = (f * D + dd) * V + AFULL * AC
                pltpu.sync_copy(tr2.at[pl.ds(dd * ATAIL, ATAIL)],
                                flat_hbm.at[pl.ds(base, ATAIL)])
                return 0

            lax.fori_loop(0, D, t_out, 0, unroll=False)

    return _detile_kernel


@functools.cache
def _build_lookup():
    mesh = plsc.VectorSubcoreMesh(core_axis_name="c", subcore_axis_name="s")

    @functools.partial(
        pl.kernel,
        mesh=mesh,
        out_type=jax.ShapeDtypeStruct((S, F, 4, B // 128, 8, 128),
                                      jnp.float32),
        scratch_types=[
            pltpu.VMEM((V,), jnp.float32),
            pltpu.VMEM((B,), jnp.int32),
            pltpu.VMEM((B,), jnp.int32),
            pltpu.VMEM((B // 128, 128), jnp.float32),
            pltpu.VMEM((B // 128, 128), jnp.float32),
            pltpu.SemaphoreType.DMA,
            pltpu.SemaphoreType.DMA,
            pltpu.SemaphoreType.DMA,
            pltpu.SemaphoreType.DMA,
        ],
        compiler_params=pltpu.CompilerParams(needs_layout_passes=False),
    )
    def _lookup_kernel(idx_hbm, tab_hbm, out_hbm, rowbuf, ib0, ib1, ob0, ob1,
                       is0, is1, os0, os1):
        ibuf, obuf = [ib0, ib1], [ob0, ob1]
        isem, osem = [is0, is1], [os0, os1]
        wid = lax.axis_index("s") * NC + lax.axis_index("c")
        d = wid
        dt, di = d // 8, d % 8

        def idx_copy(f, s, b):
            return pltpu.make_async_copy(
                idx_hbm.at[pl.ds((f * S + s) * B, B)], ibuf[b], isem[b])

        def out_cp(f, s, b):
            return pltpu.make_async_copy(
                obuf[b], out_hbm.at[s, f, dt, :, di, :], osem[b])

        def f_body(f, _):
            pltpu.sync_copy(tab_hbm.at[pl.ds((f * D + d) * V, V)], rowbuf)
            idx_copy(f, 0, 0).start()

            def s_dbl(s2, _):
                for b in range(2):
                    s = s2 * 2 + b

                    @pl.when(s + 1 < S)
                    def _prefetch():
                        idx_copy(f, s + 1, 1 - b).start()

                    idx_copy(f, s, b).wait()

                    @pl.when(s - 2 >= 0)
                    def _drain():
                        out_cp(f, s - 2, b).wait()

                    @plsc.parallel_loop(0, B // L, unroll=8)
                    def g_body(j):
                        iv = ibuf[b][pl.ds(j * L, L)]
                        x = plsc.load_gather(rowbuf, [iv])
                        obuf[b][j // 8, pl.ds((j % 8) * L, L)] = x

                    out_cp(f, s, b).start()
                return 0

            lax.fori_loop(0, S // 2, s_dbl, 0, unroll=False)
            for b in range(2):
                out_cp(f, S - 2 + b, b).wait()
            return 0

        lax.fori_loop(0, F, f_body, 0, unroll=False)

    return _lookup_kernel


def kernel(inputs, tables):
    tabt = jnp.transpose(tables, (0, 2, 1))
    tail = tables[:, AFULL * AC:, :]
    flat = _build_detile()(tabt, tail)
    idx1 = jnp.transpose(inputs, (2, 1, 0)).reshape(F * S * B)
    out6 = _build_lookup()(idx1, flat)
    return jnp.transpose(out6, (3, 5, 0, 1, 2, 4)).reshape(B, S, F, D)


# speedup vs baseline: 3.8925x; 1.0023x over previous
import functools

import jax
import jax.numpy as jnp
from jax import lax
from jax.experimental import pallas as pl
from jax.experimental.pallas import tpu as pltpu
from jax.experimental.pallas import tpu_sc as plsc

B, S, F, V, D = 4096, 20, 26, 100000, 32
L = 16
NC, NS = 2, 16
NW = NC * NS

AC = 4992
AFULL = 99840 // AC
ATAIL = V - AFULL * AC
NT = F * AFULL


@functools.cache
def _build_detile():
    mesh = plsc.VectorSubcoreMesh(core_axis_name="c", subcore_axis_name="s")

    @functools.partial(
        pl.kernel,
        mesh=mesh,
        out_type=jax.ShapeDtypeStruct((F * D * V,), jnp.float32),
        scratch_types=[
            pltpu.VMEM((AC,), jnp.float32),
            pltpu.VMEM((AC,), jnp.float32),
            pltpu.VMEM((ATAIL, D), jnp.float32),
            pltpu.VMEM((ATAIL * D,), jnp.float32),
            pltpu.SemaphoreType.DMA,
            pltpu.SemaphoreType.DMA,
            pltpu.SemaphoreType.DMA,
            pltpu.SemaphoreType.DMA,
        ],
        compiler_params=pltpu.CompilerParams(use_tc_tiling_on_sc=True,
                                             needs_layout_passes=False),
    )
    def _detile_kernel(tabt_hbm, tail_hbm, flat_hbm, c0, c1, stg2, tr2,
                       i0, i1, o0, o1):
        buf, isem, osem = [c0, c1], [i0, i1], [o0, o1]
        wid = lax.axis_index("s") * NC + lax.axis_index("c")
        d = wid

        def in_copy(t, b):
            f, c = t // AFULL, t % AFULL
            return pltpu.make_async_copy(
                tabt_hbm.at[f, d, pl.ds(c * AC, AC)], buf[b], isem[b])

        def out_copy(t, b):
            f, c = t // AFULL, t % AFULL
            base = (f * D + d) * V + c * AC
            return pltpu.make_async_copy(
                buf[b], flat_hbm.at[pl.ds(base, AC)], osem[b])

        in_copy(0, 0).start()

        def dbl_body(t2, _):
            for b in range(2):
                t = t2 * 2 + b

                @pl.when(t + 1 < NT)
                def _start_next():
                    in_copy(t + 1, 1 - b).start()

                in_copy(t, b).wait()

                @pl.when(t - 2 >= 0)
                def _drain_prev():
                    out_copy(t - 2, b).wait()

                out_copy(t, b).start()
            return 0

        lax.fori_loop(0, NT // 2, dbl_body, 0, unroll=False)
        for b in range(2):
            out_copy(NT - 2 + b, b).wait()

        @pl.when(wid < F)
        def _tail():
            f = wid
            pltpu.sync_copy(tail_hbm.at[f], stg2)
            colv = lax.iota(jnp.int32, L) * ATAIL

            def t_body(j, _):
                x0 = stg2[j, pl.ds(0, L)]
                x1 = stg2[j, pl.ds(L, L)]
                plsc.store_scatter(tr2, [colv + j], x0)
                plsc.store_scatter(
                    tr2.at[pl.ds(L * ATAIL, L * ATAIL)], [colv + j], x1)
                return 0

            lax.fori_loop(0, ATAIL, t_body, 0, unroll=False)

            def t_out(dd, _):
                base = (f * D + dd) * V + AFULL * AC
                pltpu.sync_copy(tr2.at[pl.ds(dd * ATAIL, ATAIL)],
                                flat_hbm.at[pl.ds(base, ATAIL)])
                return 0

            lax.fori_loop(0, D, t_out, 0, unroll=False)

    return _detile_kernel


@functools.cache
def _build_lookup():
    mesh = plsc.VectorSubcoreMesh(core_axis_name="c", subcore_axis_name="s")

    @functools.partial(
        pl.kernel,
        mesh=mesh,
        out_type=jax.ShapeDtypeStruct((S, F, 4, B // 128, 8, 128),
                                      jnp.float32),
        scratch_types=[
            pltpu.VMEM((V,), jnp.float32),
            pltpu.VMEM((B,), jnp.int32),
            pltpu.VMEM((B,), jnp.int32),
            pltpu.VMEM((B // 128, 128), jnp.float32),
            pltpu.VMEM((B // 128, 128), jnp.float32),
            pltpu.SemaphoreType.DMA,
            pltpu.SemaphoreType.DMA,
            pltpu.SemaphoreType.DMA,
            pltpu.SemaphoreType.DMA,
        ],
        compiler_params=pltpu.CompilerParams(needs_layout_passes=False),
    )
    def _lookup_kernel(idx_hbm, tab_hbm, out_hbm, rowbuf, ib0, ib1, ob0, ob1,
                       is0, is1, os0, os1):
        ibuf, obuf = [ib0, ib1], [ob0, ob1]
        isem, osem = [is0, is1], [os0, os1]
        wid = lax.axis_index("s") * NC + lax.axis_index("c")
        d = wid
        dt, di = d // 8, d % 8

        def idx_copy(f, s, b):
            return pltpu.make_async_copy(
                idx_hbm.at[pl.ds((f * S + s) * B, B)], ibuf[b], isem[b])

        def out_cp(f, s, b):
            return pltpu.make_async_copy(
                obuf[b], out_hbm.at[s, f, dt, :, di, :], osem[b])

        def f_body(f, _):
            pltpu.sync_copy(tab_hbm.at[pl.ds((f * D + d) * V, V)], rowbuf)
            idx_copy(f, 0, 0).start()

            def s_dbl(s2, _):
                for b in range(2):
                    s = s2 * 2 + b

                    @pl.when(s + 1 < S)
                    def _prefetch():
                        idx_copy(f, s + 1, 1 - b).start()

                    idx_copy(f, s, b).wait()

                    @pl.when(s - 2 >= 0)
                    def _drain():
                        out_cp(f, s - 2, b).wait()

                    @plsc.parallel_loop(0, B // L, unroll=16)
                    def g_body(j):
                        iv = ibuf[b][pl.ds(j * L, L)]
                        x = plsc.load_gather(rowbuf, [iv])
                        obuf[b][j // 8, pl.ds((j % 8) * L, L)] = x

                    out_cp(f, s, b).start()
                return 0

            lax.fori_loop(0, S // 2, s_dbl, 0, unroll=False)
            for b in range(2):
                out_cp(f, S - 2 + b, b).wait()
            return 0

        lax.fori_loop(0, F, f_body, 0, unroll=False)

    return _lookup_kernel


def kernel(inputs, tables):
    tabt = jnp.transpose(tables, (0, 2, 1))
    tail = tables[:, AFULL * AC:, :]
    flat = _build_detile()(tabt, tail)
    idx1 = jnp.transpose(inputs, (2, 1, 0)).reshape(F * S * B)
    out6 = _build_lookup()(idx1, flat)
    return jnp.transpose(out6, (3, 5, 0, 1, 2, 4)).reshape(B, S, F, D)
